# Initial kernel scaffold; baseline (speedup 1.0000x reference)
import jax
import jax.numpy as jnp
from jax.experimental import pallas as pl


def kernel(x, edge_index):
    raise NotImplementedError("write your pallas kernel here")


# speedup vs baseline: 8.1369x; 8.1369x over previous
import functools

import jax
import jax.numpy as jnp
from jax import lax
from jax.experimental import pallas as pl
from jax.experimental.pallas import tpu as pltpu
from jax.experimental.pallas import tpu_sc as plsc

NC, NS, LANES = 2, 16, 16
NW = NC * NS

N_NODES = 10000
D = 128
N_EDGES = 320000
NUM_ITERS = 10

NP = 10112
ROWS_PER_TILE = NP // NS
PAD_ROWS = NP - N_NODES

CHUNK = 512
CHUNK_IDX_ROWS = CHUNK // 128
NSLOTS = 2
EP = 327680
IDX_ROWS = EP // 128
IDX_ROWS_PER_W = IDX_ROWS // NW
NCHUNKS = IDX_ROWS_PER_W // CHUNK_IDX_ROWS

_mesh = plsc.VectorSubcoreMesh(core_axis_name="c", subcore_axis_name="s")


@functools.partial(
    pl.kernel,
    out_type=jax.ShapeDtypeStruct((NC, NP, D), jnp.float32),
    mesh=_mesh,
    scratch_types=[
        pltpu.VMEM_SHARED((NP, D), jnp.float32),
        pltpu.VMEM((NSLOTS * 128, D), jnp.float32),
        pltpu.VMEM((CHUNK_IDX_ROWS, 128), jnp.int32),
        pltpu.VMEM((CHUNK_IDX_ROWS, 128), jnp.int32),
        pltpu.SemaphoreType.DMA,
    ],
)
def _sc_round(h_hbm, src_hbm, dst_hbm, out_hbm, agg, rows, sidx, didx, sem):
    c = lax.axis_index("c")
    s = lax.axis_index("s")
    w = s * NC + c
    base = s * ROWS_PER_TILE

    zv = jnp.zeros((LANES,), jnp.float32)

    def zero_body(i, carry):
        for k in range(D // LANES):
            rows[i, pl.ds(k * LANES, LANES)] = zv
        return carry

    lax.fori_loop(0, 64, zero_body, 0)
    for k in range(9):
        pltpu.sync_copy(rows.at[pl.ds(0, 64)], agg.at[pl.ds(base + k * 64, 64)])
    pltpu.sync_copy(rows.at[pl.ds(0, 56)], agg.at[pl.ds(base + 576, 56)])
    plsc.subcore_barrier()

    def slot(j):
        return rows.at[pl.ds((j % NSLOTS) * 128, 128)]

    def gather(j):
        return pltpu.async_copy(h_hbm.at[sidx.at[j]], slot(j), sem)

    def chunk_body(i, carry):
        row0 = w * IDX_ROWS_PER_W + i * CHUNK_IDX_ROWS
        pltpu.sync_copy(src_hbm.at[pl.ds(row0, CHUNK_IDX_ROWS)], sidx)
        pltpu.sync_copy(dst_hbm.at[pl.ds(row0, CHUNK_IDX_ROWS)], didx)
        cp = gather(0)
        for j in range(CHUNK_IDX_ROWS):
            cp.wait()
            if j + 1 < CHUNK_IDX_ROWS:
                cp = gather(j + 1)
            pltpu.sync_copy(slot(j), agg.at[didx.at[j]], add=True)
        return carry

    lax.fori_loop(0, NCHUNKS, chunk_body, 0)
    plsc.subcore_barrier()

    pltpu.sync_copy(
        agg.at[pl.ds(base, ROWS_PER_TILE)],
        out_hbm.at[c, pl.ds(base, ROWS_PER_TILE)],
    )


_BLK = 632


def _combine_body(p0_ref, p1_ref, d0_ref, d1_ref, o_ref):
    scale = 1.0 / jnp.maximum(d0_ref[...] + d1_ref[...], 1.0)
    o_ref[...] = (p0_ref[...] + p1_ref[...]) * scale


_combine = pl.pallas_call(
    _combine_body,
    out_shape=jax.ShapeDtypeStruct((NP, D), jnp.float32),
    grid=(NP // _BLK,),
    in_specs=[
        pl.BlockSpec((_BLK, D), lambda i: (i, 0)),
        pl.BlockSpec((_BLK, D), lambda i: (i, 0)),
        pl.BlockSpec((_BLK, 1), lambda i: (i, 0)),
        pl.BlockSpec((_BLK, 1), lambda i: (i, 0)),
    ],
    out_specs=pl.BlockSpec((_BLK, D), lambda i: (i, 0)),
)


def kernel(x, edge_index):
    src = edge_index[0].astype(jnp.int32)
    dst = edge_index[1].astype(jnp.int32)
    pad = N_NODES + (lax.iota(jnp.int32, EP - N_EDGES) % PAD_ROWS)
    srcp = jnp.concatenate([src, pad]).reshape(IDX_ROWS, 128)
    dstp = jnp.concatenate([dst, pad]).reshape(IDX_ROWS, 128)
    h = jnp.concatenate(
        [x, jnp.zeros((PAD_ROWS, D), jnp.float32)], axis=0
    )

    dp = _sc_round(jnp.ones((NP, D), jnp.float32), srcp, dstp)
    d0 = dp[0, :, 0:1]
    d1 = dp[1, :, 0:1]

    for _ in range(NUM_ITERS):
        p = _sc_round(h, srcp, dstp)
        h = _combine(p[0], p[1], d0, d1)

    return h[:N_NODES]


# speedup vs baseline: 9.8853x; 1.2149x over previous
import functools

import jax
import jax.numpy as jnp
from jax import lax
from jax.experimental import pallas as pl
from jax.experimental.pallas import tpu as pltpu
from jax.experimental.pallas import tpu_sc as plsc

NC, NS, LANES = 2, 16, 16
NW = NC * NS

N_NODES = 10000
D = 128
N_EDGES = 320000
NUM_ITERS = 10

PAD_ROWS = 64
NH = N_NODES + PAD_ROWS
NA = 10112
ROWS_PER_TILE = NA // NS

CHUNK_IDX_ROWS = 3
CHUNK = CHUNK_IDX_ROWS * 128
NCHUNKS = 27
EP = NW * NCHUNKS * CHUNK
NCHUNKS_TOT = EP // CHUNK

_mesh = plsc.VectorSubcoreMesh(core_axis_name="c", subcore_axis_name="s")


@functools.partial(
    pl.kernel,
    out_type=jax.ShapeDtypeStruct((NC, NA, D), jnp.float32),
    mesh=_mesh,
    scratch_types=[
        pltpu.VMEM_SHARED((NA, D), jnp.float32),
        pltpu.VMEM((CHUNK_IDX_ROWS * 128, D), jnp.float32),
        pltpu.VMEM((2 * CHUNK_IDX_ROWS, 128), jnp.int32),
        pltpu.SemaphoreType.DMA,
    ],
)
def _sc_round(h_hbm, eidx_hbm, out_hbm, agg, rows, eidx, sem):
    c = lax.axis_index("c")
    s = lax.axis_index("s")
    w = s * NC + c
    base = s * ROWS_PER_TILE

    zv = jnp.zeros((LANES,), jnp.float32)

    def zero_body(i, carry):
        for k in range(D // LANES):
            rows[i, pl.ds(k * LANES, LANES)] = zv
        return carry

    lax.fori_loop(0, 64, zero_body, 0)
    for k in range(9):
        pltpu.sync_copy(rows.at[pl.ds(0, 64)], agg.at[pl.ds(base + k * 64, 64)])
    pltpu.sync_copy(rows.at[pl.ds(0, 56)], agg.at[pl.ds(base + 576, 56)])
    plsc.subcore_barrier()

    def slot(j):
        return rows.at[pl.ds(j * 128, 128)]

    def gather(j):
        return pltpu.async_copy(h_hbm.at[eidx.at[j]], slot(j), sem)

    def chunk_body(i, carry):
        pltpu.sync_copy(eidx_hbm.at[w * NCHUNKS + i], eidx)
        cps = [gather(0), gather(1)]
        for j in range(CHUNK_IDX_ROWS):
            cps[j].wait()
            if j + 2 < CHUNK_IDX_ROWS:
                cps.append(gather(j + 2))
            pltpu.sync_copy(
                slot(j), agg.at[eidx.at[CHUNK_IDX_ROWS + j]], add=True
            )
        return carry

    lax.fori_loop(0, NCHUNKS, chunk_body, 0)
    plsc.subcore_barrier()

    pltpu.sync_copy(
        agg.at[pl.ds(base, ROWS_PER_TILE)],
        out_hbm.at[c, pl.ds(base, ROWS_PER_TILE)],
    )


def _combine_body(p0_ref, p1_ref, d0_ref, d1_ref, o_ref):
    scale = 1.0 / jnp.maximum(d0_ref[...] + d1_ref[...], 1.0)
    o_ref[pl.ds(0, N_NODES), :] = (
        p0_ref[pl.ds(0, N_NODES), :] + p1_ref[pl.ds(0, N_NODES), :]
    ) * scale
    o_ref[pl.ds(N_NODES, PAD_ROWS), :] = jnp.zeros(
        (PAD_ROWS, D), jnp.float32
    )


_combine = pl.pallas_call(
    _combine_body,
    out_shape=jax.ShapeDtypeStruct((NH, D), jnp.float32),
)


def kernel(x, edge_index):
    src = edge_index[0].astype(jnp.int32)
    dst = edge_index[1].astype(jnp.int32)
    npad = EP - N_EDGES
    pad_src = N_NODES + (lax.iota(jnp.int32, npad) % PAD_ROWS)
    pad_dst = lax.iota(jnp.int32, npad) % N_NODES
    srcp = jnp.concatenate([src, pad_src]).reshape(NCHUNKS_TOT, 3, 128)
    dstp = jnp.concatenate([dst, pad_dst]).reshape(NCHUNKS_TOT, 3, 128)
    eidx = jnp.concatenate([srcp, dstp], axis=1)

    h = jnp.concatenate([x, jnp.zeros((PAD_ROWS, D), jnp.float32)], axis=0)

    ones_h = jnp.concatenate(
        [
            jnp.ones((N_NODES, D), jnp.float32),
            jnp.zeros((PAD_ROWS, D), jnp.float32),
        ],
        axis=0,
    )
    dp = _sc_round(ones_h, eidx)
    d0 = dp[0, :N_NODES, 0:1]
    d1 = dp[1, :N_NODES, 0:1]

    for _ in range(NUM_ITERS):
        p = _sc_round(h, eidx)
        h = _combine(p[0], p[1], d0, d1)

    return h[:N_NODES]


# speedup vs baseline: 13.2776x; 1.3432x over previous
import functools

import jax
import jax.numpy as jnp
from jax import lax
from jax.experimental import pallas as pl
from jax.experimental.pallas import tpu as pltpu
from jax.experimental.pallas import tpu_sc as plsc

NC, NS, LANES = 2, 16, 16
NW = NC * NS

N_NODES = 10000
D = 128
N_EDGES = 320000
NUM_ITERS = 10

PAD_ROWS = 64
NH = N_NODES + PAD_ROWS
NA = 10112
ROWS_PER_TILE = NA // NS

CHUNK = 120
NCHUNKS = 84
EP = NW * NCHUNKS * CHUNK

_mesh = plsc.VectorSubcoreMesh(core_axis_name="c", subcore_axis_name="s")


@functools.partial(
    pl.kernel,
    out_type=jax.ShapeDtypeStruct((NC, NA, D), jnp.float32),
    mesh=_mesh,
    scratch_types=[
        pltpu.VMEM_SHARED((NA, D), jnp.float32),
        pltpu.VMEM((3 * CHUNK, D), jnp.float32),
        [pltpu.VMEM((CHUNK,), jnp.int32)] * 6,
        [pltpu.VMEM((CHUNK,), jnp.int32)] * 6,
        [pltpu.SemaphoreType.DMA] * 3,
        [pltpu.SemaphoreType.DMA] * 3,
        [pltpu.SemaphoreType.DMA] * 6,
    ],
)
def _sc_round(h_hbm, src_hbm, dst_hbm, out_hbm, agg, rows, sidx, didx,
              sem_g, sem_s, sem_i):
    c = lax.axis_index("c")
    s = lax.axis_index("s")
    w = s * NC + c
    base = s * ROWS_PER_TILE

    slots = [rows.at[pl.ds(k * CHUNK, CHUNK)] for k in range(3)]

    zv = jnp.zeros((LANES,), jnp.float32)

    def zero_body(i, carry):
        for k in range(D // LANES):
            rows[i, pl.ds(k * LANES, LANES)] = zv
        return carry

    lax.fori_loop(0, 64, zero_body, 0)
    for k in range(9):
        pltpu.sync_copy(rows.at[pl.ds(0, 64)], agg.at[pl.ds(base + k * 64, 64)])
    pltpu.sync_copy(rows.at[pl.ds(0, 56)], agg.at[pl.ds(base + 576, 56)])
    plsc.subcore_barrier()

    def off(i):
        return (w * NCHUNKS + i) * CHUNK

    def load_idx(i, q):
        pltpu.async_copy(src_hbm.at[pl.ds(off(i), CHUNK)], sidx[q], sem_i[q])
        pltpu.async_copy(dst_hbm.at[pl.ds(off(i), CHUNK)], didx[q], sem_i[q])

    def gather(q, p):
        pltpu.async_copy(h_hbm.at[sidx[q]], slots[p], sem_g[p])

    def scatter(q, p):
        pltpu.async_copy(slots[p], agg.at[didx[q]], sem_s[p], add=True)

    def drain_g(p):
        pltpu.make_async_copy(
            h_hbm.at[pl.ds(0, CHUNK)], slots[p], sem_g[p]
        ).wait()

    def drain_s(p):
        pltpu.make_async_copy(
            h_hbm.at[pl.ds(0, CHUNK)], slots[p], sem_s[p]
        ).wait()

    def drain_i(q):
        for _ in range(2):
            pltpu.make_async_copy(
                src_hbm.at[pl.ds(0, CHUNK)], sidx[q], sem_i[q]
            ).wait()

    def stage(i, jmod, first_s=False, idx_next=True, gather_next=True):
        if not first_s:
            drain_s((jmod + 2) % 3)
        if gather_next:
            drain_i((jmod + 2) % 6)
            gather((jmod + 2) % 6, (jmod + 2) % 3)
        drain_g(jmod % 3)
        if idx_next:
            load_idx(i + 4, (jmod + 4) % 6)
        scatter(jmod, jmod % 3)

    pltpu.sync_copy(src_hbm.at[pl.ds(off(0), CHUNK)], sidx[0])
    pltpu.sync_copy(dst_hbm.at[pl.ds(off(0), CHUNK)], didx[0])
    gather(0, 0)
    pltpu.sync_copy(src_hbm.at[pl.ds(off(1), CHUNK)], sidx[1])
    pltpu.sync_copy(dst_hbm.at[pl.ds(off(1), CHUNK)], didx[1])
    gather(1, 1)
    load_idx(2, 2)
    load_idx(3, 3)
    stage(0, 0, first_s=True)
    stage(1, 1)

    def loop_body(k, carry):
        for jj in range(6):
            stage(2 + 6 * k + jj, (2 + jj) % 6)
        return carry

    lax.fori_loop(0, 12, loop_body, 0)

    for i in range(74, NCHUNKS):
        stage(
            i,
            i % 6,
            idx_next=(i + 4 < NCHUNKS),
            gather_next=(i + 2 < NCHUNKS),
        )
    drain_s((NCHUNKS - 1) % 3)
    plsc.subcore_barrier()

    pltpu.sync_copy(
        agg.at[pl.ds(base, ROWS_PER_TILE)],
        out_hbm.at[c, pl.ds(base, ROWS_PER_TILE)],
    )


def _combine_body(p0_ref, p1_ref, d0_ref, d1_ref, o_ref):
    scale = 1.0 / jnp.maximum(d0_ref[...] + d1_ref[...], 1.0)
    o_ref[pl.ds(0, N_NODES), :] = (
        p0_ref[pl.ds(0, N_NODES), :] + p1_ref[pl.ds(0, N_NODES), :]
    ) * scale
    o_ref[pl.ds(N_NODES, PAD_ROWS), :] = jnp.zeros(
        (PAD_ROWS, D), jnp.float32
    )


_combine = pl.pallas_call(
    _combine_body,
    out_shape=jax.ShapeDtypeStruct((NH, D), jnp.float32),
)


def kernel(x, edge_index):
    src = edge_index[0].astype(jnp.int32)
    dst = edge_index[1].astype(jnp.int32)
    npad = EP - N_EDGES
    pad_src = N_NODES + (lax.iota(jnp.int32, npad) % PAD_ROWS)
    pad_dst = lax.iota(jnp.int32, npad) % N_NODES
    srcp = jnp.concatenate([src, pad_src])
    dstp = jnp.concatenate([dst, pad_dst])

    h = jnp.concatenate([x, jnp.zeros((PAD_ROWS, D), jnp.float32)], axis=0)

    ones_h = jnp.concatenate(
        [
            jnp.ones((N_NODES, D), jnp.float32),
            jnp.zeros((PAD_ROWS, D), jnp.float32),
        ],
        axis=0,
    )
    dp = _sc_round(ones_h, srcp, dstp)
    d0 = dp[0, :N_NODES, 0:1]
    d1 = dp[1, :N_NODES, 0:1]

    for _ in range(NUM_ITERS):
        p = _sc_round(h, srcp, dstp)
        h = _combine(p[0], p[1], d0, d1)

    return h[:N_NODES]
